# Initial kernel scaffold; baseline (speedup 1.0000x reference)
import jax
import jax.numpy as jnp
from jax.experimental import pallas as pl


def kernel(x, pos_embedding):
    raise NotImplementedError("write your pallas kernel here")


# speedup vs baseline: 1.7299x; 1.7299x over previous
import jax
import jax.numpy as jnp
from jax.experimental import pallas as pl

_BATCH = 4
_SEQ = 8192
_HIDDEN = 1024
_BS = 512


def _add_body(x_ref, p_ref, o_ref):
    o_ref[...] = x_ref[...] + p_ref[...]


def kernel(x, pos_embedding):
    grid = (_SEQ // _BS,)
    return pl.pallas_call(
        _add_body,
        grid=grid,
        in_specs=[
            pl.BlockSpec((_BATCH, _BS, _HIDDEN), lambda i: (0, i, 0)),
            pl.BlockSpec((1, _BS, _HIDDEN), lambda i: (0, i, 0)),
        ],
        out_specs=pl.BlockSpec((_BATCH, _BS, _HIDDEN), lambda i: (0, i, 0)),
        out_shape=jax.ShapeDtypeStruct((_BATCH, _SEQ, _HIDDEN), jnp.float32),
    )(x, pos_embedding[None])
